# baseline (speedup 1.0000x reference)
import functools

import jax
import jax.numpy as jnp
from jax import lax
from jax.experimental import pallas as pl
from jax.experimental.pallas import tpu as pltpu
from jax.experimental.pallas import tpu_sc as plsc

K = 1024
D = 32
H = 64
LN_EPS = 1e-5
COMMIT = 0.25

NC = 2
NS = 16
NW = NC * NS
B = 8 * 8192
BPW = B // NW
CHUNK = 128
NCHUNK = BPW // CHUNK


def _tables_body(w1_ref, b1_ref, g_ref, bb_ref, w2_ref, b2_ref, cb_ref,
                 qtab_ref, pk_ref):
    f32 = jnp.float32
    vals = lax.broadcasted_iota(jnp.int32, (K, 1), 0).astype(f32) / (K - 1) * 2.0 - 1.0
    h = vals * w1_ref[...] + b1_ref[...]
    mu = jnp.mean(h, axis=1, keepdims=True)
    var = jnp.mean((h - mu) ** 2, axis=1, keepdims=True)
    h = (h - mu) / jnp.sqrt(var + LN_EPS) * g_ref[...] + bb_ref[...]
    h = jnp.maximum(h, 0.0)
    z = lax.dot_general(h, w2_ref[...], (((1,), (0,)), ((), ())),
                        preferred_element_type=f32,
                        precision=lax.Precision.HIGHEST) + b2_ref[...]
    cb = cb_ref[...]
    z2 = jnp.sum(z * z, axis=1, keepdims=True)
    c2 = jnp.sum(cb * cb, axis=1, keepdims=True)
    zc = lax.dot_general(z, cb, (((1,), (1,)), ((), ())),
                         preferred_element_type=f32,
                         precision=lax.Precision.HIGHEST)
    dist = z2 - 2.0 * zc + c2.reshape(1, K)
    rowmin = jnp.min(dist, axis=1, keepdims=True)
    col = lax.broadcasted_iota(jnp.int32, (K, K), 1)
    lut = jnp.min(jnp.where(dist == rowmin, col, K), axis=1).astype(jnp.int32)
    onehot = (col == lut[:, None]).astype(f32)
    qtab = lax.dot_general(onehot, cb, (((1,), (0,)), ((), ())),
                           preferred_element_type=f32)
    qtab_ref[...] = qtab
    d2bits = lax.bitcast_convert_type(rowmin.reshape(8, K // 8), jnp.int32)
    d2r = jax.lax.bitwise_and(d2bits + 0x8000, jnp.int32(-65536))
    pk_ref[...] = jax.lax.bitwise_or(d2r, lut.reshape(8, K // 8))


def _build_tables(W1, b1, ln_g, ln_b, W2, b2, codebook):
    return pl.pallas_call(
        _tables_body,
        out_shape=[
            jax.ShapeDtypeStruct((K, D), jnp.float32),
            jax.ShapeDtypeStruct((8, K // 8), jnp.int32),
        ],
    )(W1, b1.reshape(1, H), ln_g.reshape(1, H), ln_b.reshape(1, H),
      W2, b2.reshape(1, D), codebook)


def _sc_body(tf_hbm, qtab_hbm, pk_hbm,
             q_hbm, idxo_hbm, part_hbm,
             idx_v, rows_v, pk_v, idxo_v, acc_v,
             sem_q, sem_w, sem_t):
    wid = lax.axis_index("s") * NC + lax.axis_index("c")
    b = wid // (8192 // BPW)
    off = (wid % (8192 // BPW)) * BPW
    tcopy = pltpu.async_copy(tf_hbm.at[pl.ds(wid * NCHUNK, NCHUNK)], idx_v, sem_t)
    pltpu.sync_copy(pk_hbm, pk_v)
    tcopy.wait()
    gathers = []
    for j in range(NCHUNK):
        gathers.append(pltpu.async_copy(
            qtab_hbm.at[idx_v.at[j]],
            rows_v.at[pl.ds(j * CHUNK, CHUNK)], sem_q))
    acc = jnp.zeros((16,), jnp.float32)
    for j in range(NCHUNK):
        for g in range(CHUNK // 16):
            toks = idx_v[j, pl.ds(g * 16, 16)]
            w = plsc.load_gather(pk_v, [toks])
            idxo_v[j, pl.ds(g * 16, 16)] = jax.lax.bitwise_and(w, 65535)
            acc = acc + lax.bitcast_convert_type(
                jax.lax.bitwise_and(w, jnp.int32(-65536)), jnp.float32)
    acc_v[...] = acc
    writes = []
    for j in range(NCHUNK):
        writes.append(pltpu.async_copy(
            idxo_v.at[j], idxo_hbm.at[b, pl.ds(off + j * CHUNK, CHUNK)], sem_w))
    pltpu.sync_copy(acc_v, part_hbm.at[wid])
    for j in range(NCHUNK):
        gathers[j].wait()
        writes.append(pltpu.async_copy(
            rows_v.at[pl.ds(j * CHUNK, CHUNK)],
            q_hbm.at[b, pl.ds(off + j * CHUNK, CHUNK)], sem_w))
    for w in writes:
        w.wait()


@functools.lru_cache(maxsize=1)
def _make_sc_gather():
    return functools.partial(
        pl.kernel,
        out_type=[
            jax.ShapeDtypeStruct((8, 8192, D), jnp.float32),
            jax.ShapeDtypeStruct((8, 8192), jnp.int32),
            jax.ShapeDtypeStruct((NW, 16), jnp.float32),
        ],
        mesh=plsc.VectorSubcoreMesh(core_axis_name="c", subcore_axis_name="s",
                                    num_cores=NC, num_subcores=NS),
        scratch_types=[
            pltpu.VMEM((NCHUNK, CHUNK), jnp.int32),
            pltpu.VMEM((BPW, D), jnp.float32),
            pltpu.VMEM((K,), jnp.int32),
            pltpu.VMEM((NCHUNK, CHUNK), jnp.int32),
            pltpu.VMEM((16,), jnp.float32),
            pltpu.SemaphoreType.DMA,
            pltpu.SemaphoreType.DMA,
            pltpu.SemaphoreType.DMA,
        ],
        compiler_params=pltpu.CompilerParams(use_tc_tiling_on_sc=False,
                                             needs_layout_passes=False),
    )(_sc_body)


def kernel(t, W1, b1, ln_g, ln_b, W2, b2, codebook):
    qtab, pk8 = _build_tables(W1, b1, ln_g, ln_b, W2, b2, codebook)
    tf = t.reshape(B // CHUNK, CHUNK)
    q, idx, partials = _make_sc_gather()(tf, qtab, pk8.reshape(K))
    loss = (1.0 + COMMIT) * jnp.sum(partials) / (B * D)
    return q, idx, loss


# speedup vs baseline: 1.1934x; 1.1934x over previous
import functools

import jax
import jax.numpy as jnp
from jax import lax
from jax.experimental import pallas as pl
from jax.experimental.pallas import tpu as pltpu
from jax.experimental.pallas import tpu_sc as plsc

K = 1024
D = 32
H = 64
LN_EPS = 1e-5
COMMIT = 0.25

NC = 2
NS = 16
NW = NC * NS
B = 8 * 8192
BPW = B // NW
CHUNK = 128
NCHUNK = BPW // CHUNK


def _tables_body(w1_ref, b1_ref, g_ref, bb_ref, w2_ref, b2_ref, cb_ref,
                 qtab_ref, lut_ref, d2_ref):
    f32 = jnp.float32
    vals = lax.broadcasted_iota(jnp.int32, (K, 1), 0).astype(f32) / (K - 1) * 2.0 - 1.0
    h = vals * w1_ref[...] + b1_ref[...]
    mu = jnp.mean(h, axis=1, keepdims=True)
    var = jnp.mean((h - mu) ** 2, axis=1, keepdims=True)
    h = (h - mu) / jnp.sqrt(var + LN_EPS) * g_ref[...] + bb_ref[...]
    h = jnp.maximum(h, 0.0)
    z = lax.dot_general(h, w2_ref[...], (((1,), (0,)), ((), ())),
                        preferred_element_type=f32,
                        precision=lax.Precision.HIGHEST) + b2_ref[...]
    cb = cb_ref[...]
    z2 = jnp.sum(z * z, axis=1, keepdims=True)
    c2 = jnp.sum(cb * cb, axis=1, keepdims=True)
    zc = lax.dot_general(z, cb, (((1,), (1,)), ((), ())),
                         preferred_element_type=f32,
                         precision=lax.Precision.HIGHEST)
    dist = z2 - 2.0 * zc + c2.reshape(1, K)
    rowmin = jnp.min(dist, axis=1, keepdims=True)
    col = lax.broadcasted_iota(jnp.int32, (K, K), 1)
    lut = jnp.min(jnp.where(dist == rowmin, col, K), axis=1).astype(jnp.int32)
    onehot = (col == lut[:, None]).astype(f32)
    qtab = lax.dot_general(onehot, cb, (((1,), (0,)), ((), ())),
                           preferred_element_type=f32)
    qtab_ref[...] = qtab
    lut_ref[...] = lut.reshape(8, K // 8)
    d2_ref[...] = rowmin.reshape(8, K // 8)


def _build_tables(W1, b1, ln_g, ln_b, W2, b2, codebook):
    return pl.pallas_call(
        _tables_body,
        out_shape=[
            jax.ShapeDtypeStruct((K, D), jnp.float32),
            jax.ShapeDtypeStruct((8, K // 8), jnp.int32),
            jax.ShapeDtypeStruct((8, K // 8), jnp.float32),
        ],
    )(W1, b1.reshape(1, H), ln_g.reshape(1, H), ln_b.reshape(1, H),
      W2, b2.reshape(1, D), codebook)


def _sc_body(tf_hbm, qtab_hbm, lut_hbm, d2_hbm,
             q_hbm, idxo_hbm, part_hbm,
             idx_v, rows_v, lut_v, d2_v, idxo_v, acc_v, qsh,
             sem_q, sem_w, sem_t):
    wid = lax.axis_index("s") * NC + lax.axis_index("c")
    b = wid // (8192 // BPW)
    off = (wid % (8192 // BPW)) * BPW
    tcopy = pltpu.async_copy(tf_hbm.at[pl.ds(wid * NCHUNK, NCHUNK)], idx_v, sem_t)
    @pl.when(lax.axis_index("s") == 0)
    def _():
        pltpu.sync_copy(qtab_hbm, qsh)
    pltpu.sync_copy(lut_hbm, lut_v)
    pltpu.sync_copy(d2_hbm, d2_v)
    tcopy.wait()
    plsc.subcore_barrier()
    gathers = []
    for j in range(NCHUNK):
        gathers.append(pltpu.async_copy(
            qsh.at[idx_v.at[j]],
            rows_v.at[pl.ds(j * CHUNK, CHUNK)], sem_q))
    acc = jnp.zeros((16,), jnp.float32)
    for j in range(NCHUNK):
        for g in range(CHUNK // 16):
            toks = idx_v[j, pl.ds(g * 16, 16)]
            idxo_v[j, pl.ds(g * 16, 16)] = plsc.load_gather(lut_v, [toks])
            acc = acc + plsc.load_gather(d2_v, [toks])
    acc_v[...] = acc
    writes = []
    for j in range(NCHUNK):
        gathers[j].wait()
        writes.append(pltpu.async_copy(
            rows_v.at[pl.ds(j * CHUNK, CHUNK)],
            q_hbm.at[b, pl.ds(off + j * CHUNK, CHUNK)], sem_w))
        writes.append(pltpu.async_copy(
            idxo_v.at[j], idxo_hbm.at[b, pl.ds(off + j * CHUNK, CHUNK)], sem_w))
    pltpu.sync_copy(acc_v, part_hbm.at[wid])
    for w in writes:
        w.wait()


@functools.lru_cache(maxsize=1)
def _make_sc_gather():
    return functools.partial(
        pl.kernel,
        out_type=[
            jax.ShapeDtypeStruct((8, 8192, D), jnp.float32),
            jax.ShapeDtypeStruct((8, 8192), jnp.int32),
            jax.ShapeDtypeStruct((NW, 16), jnp.float32),
        ],
        mesh=plsc.VectorSubcoreMesh(core_axis_name="c", subcore_axis_name="s",
                                    num_cores=NC, num_subcores=NS),
        scratch_types=[
            pltpu.VMEM((NCHUNK, CHUNK), jnp.int32),
            pltpu.VMEM((BPW, D), jnp.float32),
            pltpu.VMEM((K,), jnp.int32),
            pltpu.VMEM((K,), jnp.float32),
            pltpu.VMEM((NCHUNK, CHUNK), jnp.int32),
            pltpu.VMEM((16,), jnp.float32),
            pltpu.VMEM_SHARED((K, D), jnp.float32),
            pltpu.SemaphoreType.DMA,
            pltpu.SemaphoreType.DMA,
            pltpu.SemaphoreType.DMA,
        ],
        compiler_params=pltpu.CompilerParams(use_tc_tiling_on_sc=False,
                                             needs_layout_passes=False),
    )(_sc_body)


def kernel(t, W1, b1, ln_g, ln_b, W2, b2, codebook):
    qtab, lut8, d28 = _build_tables(W1, b1, ln_g, ln_b, W2, b2, codebook)
    lut = lut8.reshape(K)
    d2 = d28.reshape(K)
    tf = t.reshape(B // CHUNK, CHUNK)
    q, idx, partials = _make_sc_gather()(tf, qtab, lut, d2)
    loss = (1.0 + COMMIT) * jnp.sum(partials) / (B * D)
    return q, idx, loss
